# Initial kernel scaffold; baseline (speedup 1.0000x reference)
import jax
import jax.numpy as jnp
from jax.experimental import pallas as pl


def kernel(x, edge_index, W1, b1, W2, b2, W3, b3, W4, b4, W5, b5):
    raise NotImplementedError("write your pallas kernel here")


# speedup vs baseline: 17.6366x; 17.6366x over previous
import functools

import jax
import jax.numpy as jnp
from jax import lax
from jax.experimental import pallas as pl
from jax.experimental.pallas import tpu as pltpu
from jax.experimental.pallas import tpu_sc as plsc

N = 10000
E = 320000
NC = 2
NS = 16
NW = NC * NS
EW = E // NW
K = 80
NCHUNK = EW // K
NP = 10240
RPS = NP // NS
ZROWS = 128
R = 1000


def _sc_mesh():
    return plsc.VectorSubcoreMesh(core_axis_name="c", subcore_axis_name="s")


def _deg_partials(dst3, ones_blk):
    @functools.partial(
        pl.kernel,
        mesh=_sc_mesh(),
        compiler_params=pltpu.CompilerParams(use_tc_tiling_on_sc=False),
        out_type=jax.ShapeDtypeStruct((NC, NP, 16), jnp.float32),
        scratch_types=[
            pltpu.VMEM((NCHUNK, K), jnp.int32),
            pltpu.VMEM((K, 16), jnp.float32),
            pltpu.VMEM_SHARED((NP, 16), jnp.float32),
        ],
    )
    def k(dst_hbm, ones_hbm, z_hbm, out_hbm, idx_d, rows, acc):
        c = lax.axis_index("c")
        s = lax.axis_index("s")
        w = c * NS + s
        for r in range(RPS // ZROWS):
            pltpu.sync_copy(z_hbm, acc.at[pl.ds((s * (RPS // ZROWS) + r) * ZROWS, ZROWS)])
        pltpu.sync_copy(ones_hbm, rows)
        pltpu.sync_copy(dst_hbm.at[w], idx_d)
        plsc.subcore_barrier()

        def body(j, carry):
            pltpu.sync_copy(rows, acc.at[idx_d.at[j]], add=True)
            return carry

        lax.fori_loop(0, NCHUNK, body, 0)
        plsc.subcore_barrier()
        pltpu.sync_copy(acc.at[pl.ds(s * RPS, RPS)], out_hbm.at[c, pl.ds(s * RPS, RPS)])

    return k(dst3, ones_blk, jnp.zeros((ZROWS, 16), jnp.float32))


def _edge_scatter(g, src3, dst3, fo):
    @functools.partial(
        pl.kernel,
        mesh=_sc_mesh(),
        compiler_params=pltpu.CompilerParams(use_tc_tiling_on_sc=False),
        out_type=jax.ShapeDtypeStruct((NC, NP, fo), jnp.float32),
        scratch_types=[
            pltpu.VMEM((NCHUNK, K), jnp.int32),
            pltpu.VMEM((NCHUNK, K), jnp.int32),
            pltpu.VMEM((K, fo), jnp.float32),
            pltpu.VMEM_SHARED((NP, fo), jnp.float32),
            pltpu.SemaphoreType.DMA,
        ],
    )
    def k(g_hbm, src_hbm, dst_hbm, z_hbm, out_hbm, idx_s, idx_d, rows, acc, sem):
        c = lax.axis_index("c")
        s = lax.axis_index("s")
        w = c * NS + s
        for r in range(RPS // ZROWS):
            pltpu.sync_copy(z_hbm, acc.at[pl.ds((s * (RPS // ZROWS) + r) * ZROWS, ZROWS)])
        pltpu.sync_copy(src_hbm.at[w], idx_s)
        pltpu.sync_copy(dst_hbm.at[w], idx_d)
        plsc.subcore_barrier()

        def body(j, carry):
            pltpu.async_copy(g_hbm.at[idx_s.at[j]], rows, sem).wait()
            pltpu.sync_copy(rows, acc.at[idx_d.at[j]], add=True)
            return carry

        lax.fori_loop(0, NCHUNK, body, 0)
        plsc.subcore_barrier()
        pltpu.sync_copy(acc.at[pl.ds(s * RPS, RPS)], out_hbm.at[c, pl.ds(s * RPS, RPS)])

    return k(g, src3, dst3, jnp.zeros((ZROWS, fo), jnp.float32))


def _tc_first(degp, x, W1):
    fi, fo = W1.shape

    def body(degp_ref, x_ref, w_ref, g_ref, dinv_ref):
        deg = degp_ref[0, :, :1] + degp_ref[1, :, :1] + 1.0
        dinv = lax.rsqrt(deg)
        dinv_ref[...] = dinv
        g_ref[...] = dinv * jnp.dot(
            x_ref[...], w_ref[...], preferred_element_type=jnp.float32
        )

    return pl.pallas_call(
        body,
        grid=(N // R,),
        in_specs=[
            pl.BlockSpec((NC, R, 16), lambda i: (0, i, 0)),
            pl.BlockSpec((R, fi), lambda i: (i, 0)),
            pl.BlockSpec((fi, fo), lambda i: (0, 0)),
        ],
        out_specs=[
            pl.BlockSpec((R, fo), lambda i: (i, 0)),
            pl.BlockSpec((R, 1), lambda i: (i, 0)),
        ],
        out_shape=[
            jax.ShapeDtypeStruct((N, fo), jnp.float32),
            jax.ShapeDtypeStruct((N, 1), jnp.float32),
        ],
    )(degp, x, W1)


def _tc_mid(p, g, dinv, b_prev, W):
    fi, fo = W.shape

    def body(p_ref, g_ref, dinv_ref, b_ref, w_ref, out_ref):
        h = dinv_ref[...] * (p_ref[0] + p_ref[1] + g_ref[...]) + b_ref[...]
        act = jnp.maximum(h, 0.0)
        out_ref[...] = dinv_ref[...] * jnp.dot(
            act, w_ref[...], preferred_element_type=jnp.float32
        )

    return pl.pallas_call(
        body,
        grid=(N // R,),
        in_specs=[
            pl.BlockSpec((NC, R, fi), lambda i: (0, i, 0)),
            pl.BlockSpec((R, fi), lambda i: (i, 0)),
            pl.BlockSpec((R, 1), lambda i: (i, 0)),
            pl.BlockSpec((1, fi), lambda i: (0, 0)),
            pl.BlockSpec((fi, fo), lambda i: (0, 0)),
        ],
        out_specs=pl.BlockSpec((R, fo), lambda i: (i, 0)),
        out_shape=jax.ShapeDtypeStruct((N, fo), jnp.float32),
    )(p, g, dinv, b_prev, W)


def _tc_final(p, g, dinv, b5):
    def body(p_ref, g_ref, dinv_ref, b_ref, out_ref):
        h = p_ref[0, :, :1] + p_ref[1, :, :1] + g_ref[:, :1]
        out_ref[...] = dinv_ref[...] * h + b_ref[...]

    return pl.pallas_call(
        body,
        grid=(N // R,),
        in_specs=[
            pl.BlockSpec((NC, R, 16), lambda i: (0, i, 0)),
            pl.BlockSpec((R, 16), lambda i: (i, 0)),
            pl.BlockSpec((R, 1), lambda i: (i, 0)),
            pl.BlockSpec((1, 1), lambda i: (0, 0)),
        ],
        out_specs=pl.BlockSpec((R, 1), lambda i: (i, 0)),
        out_shape=jax.ShapeDtypeStruct((N, 1), jnp.float32),
    )(p, g, dinv, b5)


def kernel(x, edge_index, W1, b1, W2, b2, W3, b3, W4, b4, W5, b5):
    src3 = edge_index[0].reshape(NW, NCHUNK, K)
    dst3 = edge_index[1].reshape(NW, NCHUNK, K)
    ones_blk = jnp.zeros((K, 16), jnp.float32).at[:, 0].set(1.0)
    W5p = jnp.pad(W5, ((0, 0), (0, 15)))

    degp = _deg_partials(dst3, ones_blk)
    G, dinv = _tc_first(degp, x, W1)

    P = _edge_scatter(G, src3, dst3, 128)
    G2 = _tc_mid(P, G, dinv, b1.reshape(1, -1), W2)
    P = _edge_scatter(G2, src3, dst3, 64)
    G3 = _tc_mid(P, G2, dinv, b2.reshape(1, -1), W3)
    P = _edge_scatter(G3, src3, dst3, 32)
    G4 = _tc_mid(P, G3, dinv, b3.reshape(1, -1), W4)
    P = _edge_scatter(G4, src3, dst3, 16)
    G5 = _tc_mid(P, G4, dinv, b4.reshape(1, -1), W5p)
    P = _edge_scatter(G5, src3, dst3, 16)
    return _tc_final(P, G5, dinv, b5.reshape(1, 1))


# speedup vs baseline: 26.6809x; 1.5128x over previous
import functools

import jax
import jax.numpy as jnp
from jax import lax
from jax.experimental import pallas as pl
from jax.experimental.pallas import tpu as pltpu
from jax.experimental.pallas import tpu_sc as plsc

N = 10000
E = 320000
NC = 2
NS = 16
NW = NC * NS
EW = E // NW
K = 80
NCHUNK = EW // K
NP = 10240
RPS = NP // NS
ZROWS = 128
R = 1000


def _sc_mesh():
    return plsc.VectorSubcoreMesh(core_axis_name="c", subcore_axis_name="s")


def _deg_partials(dst3, ones_blk):
    B = 4

    @functools.partial(
        pl.kernel,
        mesh=_sc_mesh(),
        compiler_params=pltpu.CompilerParams(use_tc_tiling_on_sc=False),
        out_type=jax.ShapeDtypeStruct((NC, NP, 16), jnp.float32),
        scratch_types=[
            pltpu.VMEM((B, K), jnp.int32),
            pltpu.VMEM((K, 16), jnp.float32),
            pltpu.VMEM_SHARED((NP, 16), jnp.float32),
            pltpu.SemaphoreType.DMA((B,)),
            pltpu.SemaphoreType.DMA((B,)),
        ],
    )
    def k(dst_hbm, ones_hbm, z_hbm, out_hbm, ids, rows, acc, isem, ssem):
        c = lax.axis_index("c")
        s = lax.axis_index("s")
        w = c * NS + s
        for r in range(RPS // ZROWS):
            pltpu.sync_copy(z_hbm, acc.at[pl.ds((s * (RPS // ZROWS) + r) * ZROWS, ZROWS)])
        pltpu.sync_copy(ones_hbm, rows)
        plsc.subcore_barrier()

        def body(j, carry):
            b0 = lax.rem(j, B)

            @pl.when(j < NCHUNK)
            def _():
                @pl.when(j >= B)
                def _():
                    pltpu.make_async_copy(
                        rows, acc.at[ids.at[b0]], ssem.at[b0]
                    ).wait()

                pltpu.async_copy(dst_hbm.at[w, j], ids.at[b0], isem.at[b0])

            @pl.when(j >= 1)
            def _():
                b1 = lax.rem(j - 1, B)
                pltpu.make_async_copy(
                    dst_hbm.at[w, 0], ids.at[b1], isem.at[b1]
                ).wait()
                pltpu.async_copy(
                    rows, acc.at[ids.at[b1]], ssem.at[b1], add=True
                )
            return carry

        lax.fori_loop(0, NCHUNK + 1, body, 0)
        for b in range(B):
            pltpu.make_async_copy(rows, acc.at[ids.at[b]], ssem.at[b]).wait()
        plsc.subcore_barrier()
        pltpu.sync_copy(acc.at[pl.ds(s * RPS, RPS)], out_hbm.at[c, pl.ds(s * RPS, RPS)])

    return k(dst3, ones_blk, jnp.zeros((ZROWS, 16), jnp.float32))


def _edge_scatter(g, eidx, fo):
    B = 3 if fo == 128 else 4

    @functools.partial(
        pl.kernel,
        mesh=_sc_mesh(),
        compiler_params=pltpu.CompilerParams(use_tc_tiling_on_sc=False),
        out_type=jax.ShapeDtypeStruct((NC, NP, fo), jnp.float32),
        scratch_types=[
            pltpu.VMEM((B, 2, K), jnp.int32),
            pltpu.VMEM((B, K, fo), jnp.float32),
            pltpu.VMEM_SHARED((NP, fo), jnp.float32),
            pltpu.SemaphoreType.DMA((B,)),
            pltpu.SemaphoreType.DMA((B,)),
            pltpu.SemaphoreType.DMA((B,)),
        ],
    )
    def k(g_hbm, ei_hbm, z_hbm, out_hbm, ids, rows, acc, isem, gsem, ssem):
        c = lax.axis_index("c")
        s = lax.axis_index("s")
        w = c * NS + s
        for r in range(RPS // ZROWS):
            pltpu.sync_copy(z_hbm, acc.at[pl.ds((s * (RPS // ZROWS) + r) * ZROWS, ZROWS)])
        plsc.subcore_barrier()

        def body(j, carry):
            b0 = lax.rem(j, B)

            @pl.when(j < NCHUNK)
            def _():
                @pl.when(j >= B)
                def _():
                    pltpu.make_async_copy(
                        rows.at[b0], acc.at[ids.at[b0, 1]], ssem.at[b0]
                    ).wait()

                pltpu.async_copy(ei_hbm.at[w, j], ids.at[b0], isem.at[b0])

            @pl.when((j >= 1) & (j <= NCHUNK))
            def _():
                b1 = lax.rem(j - 1, B)
                pltpu.make_async_copy(
                    ei_hbm.at[w, 0], ids.at[b1], isem.at[b1]
                ).wait()
                pltpu.async_copy(g_hbm.at[ids.at[b1, 0]], rows.at[b1], gsem.at[b1])

            @pl.when(j >= 2)
            def _():
                b2 = lax.rem(j - 2, B)
                pltpu.make_async_copy(
                    g_hbm.at[ids.at[b2, 0]], rows.at[b2], gsem.at[b2]
                ).wait()
                pltpu.async_copy(
                    rows.at[b2], acc.at[ids.at[b2, 1]], ssem.at[b2], add=True
                )
            return carry

        lax.fori_loop(0, NCHUNK + 2, body, 0)
        for b in range(B):
            pltpu.make_async_copy(
                rows.at[b], acc.at[ids.at[b, 1]], ssem.at[b]
            ).wait()
        plsc.subcore_barrier()
        pltpu.sync_copy(acc.at[pl.ds(s * RPS, RPS)], out_hbm.at[c, pl.ds(s * RPS, RPS)])

    return k(g, eidx, jnp.zeros((ZROWS, fo), jnp.float32))


def _tc_first(degp, x, W1):
    fi, fo = W1.shape

    def body(degp_ref, x_ref, w_ref, g_ref, dinv_ref):
        deg = degp_ref[0, :, :1] + degp_ref[1, :, :1] + 1.0
        dinv = lax.rsqrt(deg)
        dinv_ref[...] = dinv
        g_ref[...] = dinv * jnp.dot(
            x_ref[...], w_ref[...], preferred_element_type=jnp.float32
        )

    return pl.pallas_call(
        body,
        grid=(N // R,),
        in_specs=[
            pl.BlockSpec((NC, R, 16), lambda i: (0, i, 0)),
            pl.BlockSpec((R, fi), lambda i: (i, 0)),
            pl.BlockSpec((fi, fo), lambda i: (0, 0)),
        ],
        out_specs=[
            pl.BlockSpec((R, fo), lambda i: (i, 0)),
            pl.BlockSpec((R, 1), lambda i: (i, 0)),
        ],
        out_shape=[
            jax.ShapeDtypeStruct((N, fo), jnp.float32),
            jax.ShapeDtypeStruct((N, 1), jnp.float32),
        ],
    )(degp, x, W1)


def _tc_mid(p, g, dinv, b_prev, W):
    fi, fo = W.shape

    def body(p_ref, g_ref, dinv_ref, b_ref, w_ref, out_ref):
        h = dinv_ref[...] * (p_ref[0] + p_ref[1] + g_ref[...]) + b_ref[...]
        act = jnp.maximum(h, 0.0)
        out_ref[...] = dinv_ref[...] * jnp.dot(
            act, w_ref[...], preferred_element_type=jnp.float32
        )

    return pl.pallas_call(
        body,
        grid=(N // R,),
        in_specs=[
            pl.BlockSpec((NC, R, fi), lambda i: (0, i, 0)),
            pl.BlockSpec((R, fi), lambda i: (i, 0)),
            pl.BlockSpec((R, 1), lambda i: (i, 0)),
            pl.BlockSpec((1, fi), lambda i: (0, 0)),
            pl.BlockSpec((fi, fo), lambda i: (0, 0)),
        ],
        out_specs=pl.BlockSpec((R, fo), lambda i: (i, 0)),
        out_shape=jax.ShapeDtypeStruct((N, fo), jnp.float32),
    )(p, g, dinv, b_prev, W)


def _tc_final(p, g, dinv, b5):
    def body(p_ref, g_ref, dinv_ref, b_ref, out_ref):
        h = p_ref[0, :, :1] + p_ref[1, :, :1] + g_ref[:, :1]
        out_ref[...] = dinv_ref[...] * h + b_ref[...]

    return pl.pallas_call(
        body,
        grid=(N // R,),
        in_specs=[
            pl.BlockSpec((NC, R, 16), lambda i: (0, i, 0)),
            pl.BlockSpec((R, 16), lambda i: (i, 0)),
            pl.BlockSpec((R, 1), lambda i: (i, 0)),
            pl.BlockSpec((1, 1), lambda i: (0, 0)),
        ],
        out_specs=pl.BlockSpec((R, 1), lambda i: (i, 0)),
        out_shape=jax.ShapeDtypeStruct((N, 1), jnp.float32),
    )(p, g, dinv, b5)


def kernel(x, edge_index, W1, b1, W2, b2, W3, b3, W4, b4, W5, b5):
    src3 = edge_index[0].reshape(NW, NCHUNK, K)
    dst3 = edge_index[1].reshape(NW, NCHUNK, K)
    eidx = jnp.stack([src3, dst3], axis=2)
    ones_blk = jnp.zeros((K, 16), jnp.float32).at[:, 0].set(1.0)
    W5p = jnp.pad(W5, ((0, 0), (0, 15)))

    degp = _deg_partials(dst3, ones_blk)
    G, dinv = _tc_first(degp, x, W1)

    P = _edge_scatter(G, eidx, 128)
    G2 = _tc_mid(P, G, dinv, b1.reshape(1, -1), W2)
    P = _edge_scatter(G2, eidx, 64)
    G3 = _tc_mid(P, G2, dinv, b2.reshape(1, -1), W3)
    P = _edge_scatter(G3, eidx, 32)
    G4 = _tc_mid(P, G3, dinv, b3.reshape(1, -1), W4)
    P = _edge_scatter(G4, eidx, 16)
    G5 = _tc_mid(P, G4, dinv, b4.reshape(1, -1), W5p)
    P = _edge_scatter(G5, eidx, 16)
    return _tc_final(P, G5, dinv, b5.reshape(1, 1))


# speedup vs baseline: 29.4233x; 1.1028x over previous
import functools

import jax
import jax.numpy as jnp
from jax import lax
from jax.experimental import pallas as pl
from jax.experimental.pallas import tpu as pltpu
from jax.experimental.pallas import tpu_sc as plsc

N = 10000
E = 320000
NC = 2
NS = 16
NW = NC * NS
EW = E // NW
K = 80
NCHUNK = EW // K
NP = 10240
RPS = NP // NS
ZROWS = 128
R = 1000


def _sc_mesh():
    return plsc.VectorSubcoreMesh(core_axis_name="c", subcore_axis_name="s")


def _deg_partials(dst3, ones_blk):
    B, LS = 6, 2

    @functools.partial(
        pl.kernel,
        mesh=_sc_mesh(),
        compiler_params=pltpu.CompilerParams(use_tc_tiling_on_sc=False),
        out_type=jax.ShapeDtypeStruct((NC, NP, 16), jnp.float32),
        scratch_types=[
            pltpu.VMEM((B, K), jnp.int32),
            pltpu.VMEM((K, 16), jnp.float32),
            pltpu.VMEM_SHARED((NP, 16), jnp.float32),
            pltpu.SemaphoreType.DMA((B,)),
            pltpu.SemaphoreType.DMA((B,)),
        ],
    )
    def k(dst_hbm, ones_hbm, z_hbm, out_hbm, ids, rows, acc, isem, ssem):
        c = lax.axis_index("c")
        s = lax.axis_index("s")
        w = c * NS + s
        for r in range(RPS // ZROWS):
            pltpu.sync_copy(z_hbm, acc.at[pl.ds((s * (RPS // ZROWS) + r) * ZROWS, ZROWS)])
        pltpu.sync_copy(ones_hbm, rows)
        plsc.subcore_barrier()

        def body(j, carry):
            b0 = lax.rem(j, B)

            @pl.when(j < NCHUNK)
            def _():
                @pl.when(j >= B)
                def _():
                    pltpu.make_async_copy(
                        rows, acc.at[ids.at[b0]], ssem.at[b0]
                    ).wait()

                pltpu.async_copy(dst_hbm.at[w, j], ids.at[b0], isem.at[b0])

            @pl.when(j >= LS)
            def _():
                b1 = lax.rem(j - LS, B)
                pltpu.make_async_copy(
                    dst_hbm.at[w, 0], ids.at[b1], isem.at[b1]
                ).wait()
                pltpu.async_copy(
                    rows, acc.at[ids.at[b1]], ssem.at[b1], add=True
                )
            return carry

        lax.fori_loop(0, NCHUNK + LS, body, 0)
        for b in range(B):
            pltpu.make_async_copy(rows, acc.at[ids.at[b]], ssem.at[b]).wait()
        plsc.subcore_barrier()
        pltpu.sync_copy(acc.at[pl.ds(s * RPS, RPS)], out_hbm.at[c, pl.ds(s * RPS, RPS)])

    return k(dst3, ones_blk, jnp.zeros((ZROWS, 16), jnp.float32))


def _edge_scatter(g, eidx, fo):
    kc = 40 if fo == 128 else 80
    nch = EW // kc
    B, LG, LS = 6, 2, 4

    @functools.partial(
        pl.kernel,
        mesh=_sc_mesh(),
        compiler_params=pltpu.CompilerParams(use_tc_tiling_on_sc=False),
        out_type=jax.ShapeDtypeStruct((NC, NP, fo), jnp.float32),
        scratch_types=[
            pltpu.VMEM((B, 2, kc), jnp.int32),
            pltpu.VMEM((B, kc, fo), jnp.float32),
            pltpu.VMEM_SHARED((NP, fo), jnp.float32),
            pltpu.SemaphoreType.DMA((B,)),
            pltpu.SemaphoreType.DMA((B,)),
            pltpu.SemaphoreType.DMA((B,)),
        ],
    )
    def k(g_hbm, ei_hbm, z_hbm, out_hbm, ids, rows, acc, isem, gsem, ssem):
        c = lax.axis_index("c")
        s = lax.axis_index("s")
        w = c * NS + s
        for r in range(RPS // ZROWS):
            pltpu.sync_copy(z_hbm, acc.at[pl.ds((s * (RPS // ZROWS) + r) * ZROWS, ZROWS)])
        plsc.subcore_barrier()

        def body(j, carry):
            b0 = lax.rem(j, B)

            @pl.when(j < nch)
            def _():
                @pl.when(j >= B)
                def _():
                    pltpu.make_async_copy(
                        rows.at[b0], acc.at[ids.at[b0, 1]], ssem.at[b0]
                    ).wait()

                pltpu.async_copy(ei_hbm.at[w, j], ids.at[b0], isem.at[b0])

            @pl.when((j >= LG) & (j < nch + LG))
            def _():
                b1 = lax.rem(j - LG, B)
                pltpu.make_async_copy(
                    ei_hbm.at[w, 0], ids.at[b1], isem.at[b1]
                ).wait()
                pltpu.async_copy(g_hbm.at[ids.at[b1, 0]], rows.at[b1], gsem.at[b1])

            @pl.when(j >= LS)
            def _():
                b2 = lax.rem(j - LS, B)
                pltpu.make_async_copy(
                    g_hbm.at[ids.at[b2, 0]], rows.at[b2], gsem.at[b2]
                ).wait()
                pltpu.async_copy(
                    rows.at[b2], acc.at[ids.at[b2, 1]], ssem.at[b2], add=True
                )
            return carry

        lax.fori_loop(0, nch + LS, body, 0)
        for b in range(B):
            pltpu.make_async_copy(
                rows.at[b], acc.at[ids.at[b, 1]], ssem.at[b]
            ).wait()
        plsc.subcore_barrier()
        pltpu.sync_copy(acc.at[pl.ds(s * RPS, RPS)], out_hbm.at[c, pl.ds(s * RPS, RPS)])

    return k(g, eidx, jnp.zeros((ZROWS, fo), jnp.float32))


def _tc_first(degp, x, W1):
    fi, fo = W1.shape

    def body(degp_ref, x_ref, w_ref, g_ref, dinv_ref):
        deg = degp_ref[0, :, :1] + degp_ref[1, :, :1] + 1.0
        dinv = lax.rsqrt(deg)
        dinv_ref[...] = dinv
        g_ref[...] = dinv * jnp.dot(
            x_ref[...], w_ref[...], preferred_element_type=jnp.float32
        )

    return pl.pallas_call(
        body,
        grid=(N // R,),
        in_specs=[
            pl.BlockSpec((NC, R, 16), lambda i: (0, i, 0)),
            pl.BlockSpec((R, fi), lambda i: (i, 0)),
            pl.BlockSpec((fi, fo), lambda i: (0, 0)),
        ],
        out_specs=[
            pl.BlockSpec((R, fo), lambda i: (i, 0)),
            pl.BlockSpec((R, 1), lambda i: (i, 0)),
        ],
        out_shape=[
            jax.ShapeDtypeStruct((N, fo), jnp.float32),
            jax.ShapeDtypeStruct((N, 1), jnp.float32),
        ],
    )(degp, x, W1)


def _tc_mid(p, g, dinv, b_prev, W):
    fi, fo = W.shape

    def body(p_ref, g_ref, dinv_ref, b_ref, w_ref, out_ref):
        h = dinv_ref[...] * (p_ref[0] + p_ref[1] + g_ref[...]) + b_ref[...]
        act = jnp.maximum(h, 0.0)
        out_ref[...] = dinv_ref[...] * jnp.dot(
            act, w_ref[...], preferred_element_type=jnp.float32
        )

    return pl.pallas_call(
        body,
        grid=(N // R,),
        in_specs=[
            pl.BlockSpec((NC, R, fi), lambda i: (0, i, 0)),
            pl.BlockSpec((R, fi), lambda i: (i, 0)),
            pl.BlockSpec((R, 1), lambda i: (i, 0)),
            pl.BlockSpec((1, fi), lambda i: (0, 0)),
            pl.BlockSpec((fi, fo), lambda i: (0, 0)),
        ],
        out_specs=pl.BlockSpec((R, fo), lambda i: (i, 0)),
        out_shape=jax.ShapeDtypeStruct((N, fo), jnp.float32),
    )(p, g, dinv, b_prev, W)


def _tc_final(p, g, dinv, b5):
    def body(p_ref, g_ref, dinv_ref, b_ref, out_ref):
        h = p_ref[0, :, :1] + p_ref[1, :, :1] + g_ref[:, :1]
        out_ref[...] = dinv_ref[...] * h + b_ref[...]

    return pl.pallas_call(
        body,
        grid=(N // R,),
        in_specs=[
            pl.BlockSpec((NC, R, 16), lambda i: (0, i, 0)),
            pl.BlockSpec((R, 16), lambda i: (i, 0)),
            pl.BlockSpec((R, 1), lambda i: (i, 0)),
            pl.BlockSpec((1, 1), lambda i: (0, 0)),
        ],
        out_specs=pl.BlockSpec((R, 1), lambda i: (i, 0)),
        out_shape=jax.ShapeDtypeStruct((N, 1), jnp.float32),
    )(p, g, dinv, b5)


def kernel(x, edge_index, W1, b1, W2, b2, W3, b3, W4, b4, W5, b5):
    src3 = edge_index[0].reshape(NW, NCHUNK, K)
    dst3 = edge_index[1].reshape(NW, NCHUNK, K)
    eidx = jnp.stack([src3, dst3], axis=2)
    eidx40 = jnp.stack(
        [edge_index[0].reshape(NW, EW // 40, 40), edge_index[1].reshape(NW, EW // 40, 40)],
        axis=2,
    )
    ones_blk = jnp.zeros((K, 16), jnp.float32).at[:, 0].set(1.0)
    W5p = jnp.pad(W5, ((0, 0), (0, 15)))

    degp = _deg_partials(dst3, ones_blk)
    G, dinv = _tc_first(degp, x, W1)

    P = _edge_scatter(G, eidx40, 128)
    G2 = _tc_mid(P, G, dinv, b1.reshape(1, -1), W2)
    P = _edge_scatter(G2, eidx, 64)
    G3 = _tc_mid(P, G2, dinv, b2.reshape(1, -1), W3)
    P = _edge_scatter(G3, eidx, 32)
    G4 = _tc_mid(P, G3, dinv, b3.reshape(1, -1), W4)
    P = _edge_scatter(G4, eidx, 16)
    G5 = _tc_mid(P, G4, dinv, b4.reshape(1, -1), W5p)
    P = _edge_scatter(G5, eidx, 16)
    return _tc_final(P, G5, dinv, b5.reshape(1, 1))


# speedup vs baseline: 30.3405x; 1.0312x over previous
import functools

import jax
import jax.numpy as jnp
from jax import lax
from jax.experimental import pallas as pl
from jax.experimental.pallas import tpu as pltpu
from jax.experimental.pallas import tpu_sc as plsc

N = 10000
E = 320000
NC = 2
NS = 16
NW = NC * NS
EW = E // NW
K = 80
NCHUNK = EW // K
NP = 10240
RPS = NP // NS
ZROWS = 128
R = 1000


def _sc_mesh():
    return plsc.VectorSubcoreMesh(core_axis_name="c", subcore_axis_name="s")


def _deg_partials(dst3, ones_blk):
    B, LS = 6, 2

    @functools.partial(
        pl.kernel,
        mesh=_sc_mesh(),
        compiler_params=pltpu.CompilerParams(use_tc_tiling_on_sc=False),
        out_type=jax.ShapeDtypeStruct((NC, NP, 16), jnp.float32),
        scratch_types=[
            pltpu.VMEM((B, K), jnp.int32),
            pltpu.VMEM((K, 16), jnp.float32),
            pltpu.VMEM_SHARED((NP, 16), jnp.float32),
            pltpu.SemaphoreType.DMA((B,)),
            pltpu.SemaphoreType.DMA((B,)),
        ],
    )
    def k(dst_hbm, ones_hbm, z_hbm, out_hbm, ids, rows, acc, isem, ssem):
        c = lax.axis_index("c")
        s = lax.axis_index("s")
        w = c * NS + s
        pltpu.sync_copy(z_hbm, acc.at[pl.ds(s * RPS, RPS)])
        pltpu.sync_copy(ones_hbm, rows)
        plsc.subcore_barrier()

        def body(j, carry):
            b0 = lax.rem(j, B)

            @pl.when(j < NCHUNK)
            def _():
                @pl.when(j >= B)
                def _():
                    pltpu.make_async_copy(
                        rows, acc.at[ids.at[b0]], ssem.at[b0]
                    ).wait()

                pltpu.async_copy(dst_hbm.at[w, j], ids.at[b0], isem.at[b0])

            @pl.when(j >= LS)
            def _():
                b1 = lax.rem(j - LS, B)
                pltpu.make_async_copy(
                    dst_hbm.at[w, 0], ids.at[b1], isem.at[b1]
                ).wait()
                pltpu.async_copy(
                    rows, acc.at[ids.at[b1]], ssem.at[b1], add=True
                )
            return carry

        lax.fori_loop(0, NCHUNK + LS, body, 0)
        for b in range(B):
            pltpu.make_async_copy(rows, acc.at[ids.at[b]], ssem.at[b]).wait()
        plsc.subcore_barrier()
        pltpu.sync_copy(acc.at[pl.ds(s * RPS, RPS)], out_hbm.at[c, pl.ds(s * RPS, RPS)])

    return k(dst3, ones_blk, jnp.zeros((RPS, 16), jnp.float32))


def _edge_scatter(g, eidx, fo):
    kc = 40 if fo == 128 else 80
    nch = EW // kc
    B, LG, LS = 6, 2, 4

    @functools.partial(
        pl.kernel,
        mesh=_sc_mesh(),
        compiler_params=pltpu.CompilerParams(use_tc_tiling_on_sc=False),
        out_type=jax.ShapeDtypeStruct((NC, NP, fo), jnp.float32),
        scratch_types=[
            pltpu.VMEM((B, 2, kc), jnp.int32),
            pltpu.VMEM((B, kc, fo), jnp.float32),
            pltpu.VMEM_SHARED((NP, fo), jnp.float32),
            pltpu.SemaphoreType.DMA((B,)),
            pltpu.SemaphoreType.DMA((B,)),
            pltpu.SemaphoreType.DMA((B,)),
        ],
    )
    def k(g_hbm, ei_hbm, z_hbm, out_hbm, ids, rows, acc, isem, gsem, ssem):
        c = lax.axis_index("c")
        s = lax.axis_index("s")
        w = c * NS + s
        pltpu.sync_copy(z_hbm, acc.at[pl.ds(s * RPS, RPS)])
        plsc.subcore_barrier()

        def body(j, carry):
            b0 = lax.rem(j, B)

            @pl.when(j < nch)
            def _():
                @pl.when(j >= B)
                def _():
                    pltpu.make_async_copy(
                        rows.at[b0], acc.at[ids.at[b0, 1]], ssem.at[b0]
                    ).wait()

                pltpu.async_copy(ei_hbm.at[w, j], ids.at[b0], isem.at[b0])

            @pl.when((j >= LG) & (j < nch + LG))
            def _():
                b1 = lax.rem(j - LG, B)
                pltpu.make_async_copy(
                    ei_hbm.at[w, 0], ids.at[b1], isem.at[b1]
                ).wait()
                pltpu.async_copy(g_hbm.at[ids.at[b1, 0]], rows.at[b1], gsem.at[b1])

            @pl.when(j >= LS)
            def _():
                b2 = lax.rem(j - LS, B)
                pltpu.make_async_copy(
                    g_hbm.at[ids.at[b2, 0]], rows.at[b2], gsem.at[b2]
                ).wait()
                pltpu.async_copy(
                    rows.at[b2], acc.at[ids.at[b2, 1]], ssem.at[b2], add=True
                )
            return carry

        lax.fori_loop(0, nch + LS, body, 0)
        for b in range(B):
            pltpu.make_async_copy(
                rows.at[b], acc.at[ids.at[b, 1]], ssem.at[b]
            ).wait()
        plsc.subcore_barrier()
        pltpu.sync_copy(acc.at[pl.ds(s * RPS, RPS)], out_hbm.at[c, pl.ds(s * RPS, RPS)])

    return k(g, eidx, jnp.zeros((RPS, fo), jnp.float32))


def _tc_mm(x, W1):
    fi, fo = W1.shape

    def body(x_ref, w_ref, h_ref):
        h_ref[...] = jnp.dot(x_ref[...], w_ref[...], preferred_element_type=jnp.float32)

    return pl.pallas_call(
        body,
        grid=(N // R,),
        in_specs=[
            pl.BlockSpec((R, fi), lambda i: (i, 0)),
            pl.BlockSpec((fi, fo), lambda i: (0, 0)),
        ],
        out_specs=pl.BlockSpec((R, fo), lambda i: (i, 0)),
        out_shape=jax.ShapeDtypeStruct((N, fo), jnp.float32),
    )(x, W1)


def _tc_scale(degp, h):
    fo = h.shape[1]

    def body(degp_ref, h_ref, g_ref, dinv_ref):
        deg = degp_ref[0, :, :1] + degp_ref[1, :, :1] + 1.0
        dinv = lax.rsqrt(deg)
        dinv_ref[...] = dinv
        g_ref[...] = dinv * h_ref[...]

    return pl.pallas_call(
        body,
        grid=(N // R,),
        in_specs=[
            pl.BlockSpec((NC, R, 16), lambda i: (0, i, 0)),
            pl.BlockSpec((R, fo), lambda i: (i, 0)),
        ],
        out_specs=[
            pl.BlockSpec((R, fo), lambda i: (i, 0)),
            pl.BlockSpec((R, 1), lambda i: (i, 0)),
        ],
        out_shape=[
            jax.ShapeDtypeStruct((N, fo), jnp.float32),
            jax.ShapeDtypeStruct((N, 1), jnp.float32),
        ],
    )(degp, h)


def _tc_mid(p, g, dinv, b_prev, W):
    fi, fo = W.shape

    def body(p_ref, g_ref, dinv_ref, b_ref, w_ref, out_ref):
        h = dinv_ref[...] * (p_ref[0] + p_ref[1] + g_ref[...]) + b_ref[...]
        act = jnp.maximum(h, 0.0)
        out_ref[...] = dinv_ref[...] * jnp.dot(
            act, w_ref[...], preferred_element_type=jnp.float32
        )

    return pl.pallas_call(
        body,
        grid=(N // R,),
        in_specs=[
            pl.BlockSpec((NC, R, fi), lambda i: (0, i, 0)),
            pl.BlockSpec((R, fi), lambda i: (i, 0)),
            pl.BlockSpec((R, 1), lambda i: (i, 0)),
            pl.BlockSpec((1, fi), lambda i: (0, 0)),
            pl.BlockSpec((fi, fo), lambda i: (0, 0)),
        ],
        out_specs=pl.BlockSpec((R, fo), lambda i: (i, 0)),
        out_shape=jax.ShapeDtypeStruct((N, fo), jnp.float32),
    )(p, g, dinv, b_prev, W)


def _tc_final(p, g, dinv, b5):
    def body(p_ref, g_ref, dinv_ref, b_ref, out_ref):
        h = p_ref[0, :, :1] + p_ref[1, :, :1] + g_ref[:, :1]
        out_ref[...] = dinv_ref[...] * h + b_ref[...]

    return pl.pallas_call(
        body,
        grid=(N // R,),
        in_specs=[
            pl.BlockSpec((NC, R, 16), lambda i: (0, i, 0)),
            pl.BlockSpec((R, 16), lambda i: (i, 0)),
            pl.BlockSpec((R, 1), lambda i: (i, 0)),
            pl.BlockSpec((1, 1), lambda i: (0, 0)),
        ],
        out_specs=pl.BlockSpec((R, 1), lambda i: (i, 0)),
        out_shape=jax.ShapeDtypeStruct((N, 1), jnp.float32),
    )(p, g, dinv, b5)


def kernel(x, edge_index, W1, b1, W2, b2, W3, b3, W4, b4, W5, b5):
    src3 = edge_index[0].reshape(NW, NCHUNK, K)
    dst3 = edge_index[1].reshape(NW, NCHUNK, K)
    eidx = jnp.stack([src3, dst3], axis=2)
    eidx40 = jnp.stack(
        [edge_index[0].reshape(NW, EW // 40, 40), edge_index[1].reshape(NW, EW // 40, 40)],
        axis=2,
    )
    ones_blk = jnp.zeros((K, 16), jnp.float32).at[:, 0].set(1.0)
    W5p = jnp.pad(W5, ((0, 0), (0, 15)))

    H1 = _tc_mm(x, W1)
    degp = _deg_partials(dst3, ones_blk)
    G, dinv = _tc_scale(degp, H1)

    P = _edge_scatter(G, eidx40, 128)
    G2 = _tc_mid(P, G, dinv, b1.reshape(1, -1), W2)
    P = _edge_scatter(G2, eidx, 64)
    G3 = _tc_mid(P, G2, dinv, b2.reshape(1, -1), W3)
    P = _edge_scatter(G3, eidx, 32)
    G4 = _tc_mid(P, G3, dinv, b3.reshape(1, -1), W4)
    P = _edge_scatter(G4, eidx, 16)
    G5 = _tc_mid(P, G4, dinv, b4.reshape(1, -1), W5p)
    P = _edge_scatter(G5, eidx, 16)
    return _tc_final(P, G5, dinv, b5.reshape(1, 1))
